# baseline (speedup 1.0000x reference)
import functools

import jax
import jax.numpy as jnp
from jax import lax
from jax.experimental import pallas as pl
from jax.experimental.pallas import tpu as pltpu
from jax.experimental.pallas import tpu_sc as plsc

NG = 64
DELTA = 20.0 / (NG - 1)
COEFF = -0.5 / (DELTA * DELTA)
LANES = 16
NW = 32
CHUNK = 400
TPITCH = NG + 1
OPITCH = 2 * NG + 1
NROWS = 100


@functools.lru_cache(maxsize=None)
def _build(E):
    per_w = E // NW
    n_chunks = per_w // CHUNK
    n_pairs = n_chunks // 2
    mesh = plsc.VectorSubcoreMesh(
        core_axis_name="c", subcore_axis_name="s", num_cores=2, num_subcores=16
    )

    @functools.partial(
        pl.kernel,
        out_type=jax.ShapeDtypeStruct((E, 2 * NG), jnp.float32),
        mesh=mesh,
        compiler_params=pltpu.CompilerParams(
            use_tc_tiling_on_sc=False, needs_layout_passes=False
        ),
        scratch_types=[
            pltpu.VMEM((2, CHUNK), jnp.float32),
            pltpu.VMEM((2, CHUNK), jnp.int32),
            pltpu.VMEM((NROWS * TPITCH,), jnp.float32),
            pltpu.VMEM((2, CHUNK, OPITCH), jnp.float32),
            pltpu.SemaphoreType.DMA,
            pltpu.SemaphoreType.DMA,
            pltpu.SemaphoreType.DMA,
            pltpu.SemaphoreType.DMA,
        ],
    )
    def sc_kernel(len_hbm, idx_hbm, table_hbm, out_hbm,
                  len_v, idx_v, table_v, out_v,
                  in_sem0, in_sem1, out_sem0, out_sem1):
        wid = lax.axis_index("s") * 2 + lax.axis_index("c")
        lane = lax.iota(jnp.int32, LANES)
        in_sems = [in_sem0, in_sem1]
        out_sems = [out_sem0, out_sem1]

        def in_copies(c, b):
            base = wid * per_w + c * CHUNK
            return [
                pltpu.make_async_copy(
                    len_hbm.at[pl.ds(base, CHUNK)], len_v.at[b], in_sems[b]),
                pltpu.make_async_copy(
                    idx_hbm.at[pl.ds(base, CHUNK)], idx_v.at[b], in_sems[b]),
            ]

        def out_copy(c, b):
            base = wid * per_w + c * CHUNK
            return pltpu.make_async_copy(
                out_v.at[b].at[:, pl.ds(0, 2 * NG)],
                out_hbm.at[pl.ds(base, CHUNK)],
                out_sems[b])

        pltpu.sync_copy(table_hbm, table_v)
        for cp in in_copies(0, 0):
            cp.start()

        def compute(c, b):
            def e_body(e, carry2):
                d16 = len_v[b, pl.ds(e * LANES, LANES)]
                a16 = idx_v[b, pl.ds(e * LANES, LANES)] * TPITCH
                row = lane + e * LANES
                for g in range(NG):
                    t = d16 - (g * DELTA)
                    v = jnp.exp(COEFF * (t * t))
                    plsc.store_scatter(
                        out_v.at[b], [row, jnp.full((LANES,), g, jnp.int32)], v)
                    ev = plsc.load_gather(table_v, [a16 + g])
                    plsc.store_scatter(
                        out_v.at[b],
                        [row, jnp.full((LANES,), NG + g, jnp.int32)], ev)
                return carry2

            lax.fori_loop(0, CHUNK // LANES, e_body, 0, unroll=False)

        def process(c, b, prefetch, drain_out):
            @pl.when(prefetch)
            def _():
                for cp in in_copies(c + 1, 1 - b):
                    cp.start()
            for cp in in_copies(c, b):
                cp.wait()
            @pl.when(drain_out)
            def _():
                out_copy(c, b).wait()
            compute(c, b)
            out_copy(c, b).start()

        def pair(i, carry):
            c0 = 2 * i
            process(c0, 0, c0 + 1 < n_chunks, i > 0)
            process(c0 + 1, 1, c0 + 2 < n_chunks, i > 0)
            return carry

        lax.fori_loop(0, n_pairs, pair, 0, unroll=False)
        if n_chunks % 2:
            c = n_chunks - 1
            process(c, 0, False, True)
            out_copy(c, 0).wait()
            out_copy(n_chunks - 2, 1).wait()
        else:
            out_copy(n_chunks - 2, 0).wait()
            out_copy(n_chunks - 1, 1).wait()

    return sc_kernel


def kernel(edge_length, edge_type, bond_emb_weight):
    E = edge_length.shape[0]
    lengths = edge_length.reshape(E)
    idx = edge_type.astype(jnp.int32)
    table = jnp.concatenate(
        [bond_emb_weight,
         jnp.zeros((bond_emb_weight.shape[0], TPITCH - NG), jnp.float32)],
        axis=1).reshape(-1)
    fn = _build(E)
    return fn(lengths, idx, table)
